# baseline (speedup 1.0000x reference)
import functools

import jax
import jax.numpy as jnp
import numpy as np
from jax.experimental import pallas as pl

_ENUMS = (1, 2, 3, 4, 5)
_OUT = 73
_PAD = 128
_ROWS_PER_BLOCK = 4096
_NSPLIT = 1


def _mm_kernel(*refs):
    x_refs = refs[:_NSPLIT]
    w_refs = refs[_NSPLIT:2 * _NSPLIT]
    b_ref, dv_ref, idx_ref, o_ref = refs[2 * _NSPLIT:]
    dn = (((1,), (1,)), ((), ()))
    acc = jax.lax.dot_general(x_refs[0][...], w_refs[0][...], dn,
                              preferred_element_type=jnp.float32)
    for j in range(1, _NSPLIT):
        acc = acc + jax.lax.dot_general(x_refs[j][...], w_refs[j][...], dn,
                                        preferred_element_type=jnp.float32)
    acc = acc + b_ref[...]
    idxp = idx_ref[...]
    idx3 = jax.lax.broadcast_in_dim(
        idxp, (_ROWS_PER_BLOCK // _PAD, _PAD, _PAD), (0, 1))
    idx_col = idx3.reshape(_ROWS_PER_BLOCK, _PAD)
    mask = idx_col == dv_ref[...]
    o_ref[...] = jnp.where(mask, acc, 0.0)[:, :_OUT]


@jax.jit
def kernel(output_latents, output_decoder_index,
           W0, b0, W1, b1, W2, b2, W3, b3, W4, b4):
    B, T, D = output_latents.shape
    R = B * T
    Dc = D // _NSPLIT
    x = output_latents.reshape(R, D)
    idx = output_decoder_index.reshape(R // _PAD, _PAD)

    Wcat = jnp.concatenate([W0, W1, W2, W3, W4], axis=0)
    Wp = jnp.zeros((_PAD, D), jnp.float32).at[:_OUT].set(Wcat)
    bcat = jnp.concatenate([b0, b1, b2, b3, b4], axis=0)
    bp = jnp.zeros((1, _PAD), jnp.float32).at[0, :_OUT].set(bcat)

    dims = [2, 2, 2, 3, 64]
    dv_np = np.full((1, _PAD), -1, np.int32)
    off = 0
    for e, d in zip(_ENUMS, dims):
        dv_np[0, off:off + d] = e
        off += d
    dv = jnp.asarray(dv_np)

    grid = (R // _ROWS_PER_BLOCK,)
    x_specs = [
        pl.BlockSpec((_ROWS_PER_BLOCK, Dc),
                     functools.partial(lambda j, i: (i, j), j))
        for j in range(_NSPLIT)
    ]
    w_specs = [
        pl.BlockSpec((_PAD, Dc),
                     functools.partial(lambda j, i: (0, j), j))
        for j in range(_NSPLIT)
    ]
    out = pl.pallas_call(
        _mm_kernel,
        grid=grid,
        in_specs=x_specs + w_specs + [
            pl.BlockSpec((1, _PAD), lambda i: (0, 0)),
            pl.BlockSpec((1, _PAD), lambda i: (0, 0)),
            pl.BlockSpec((_ROWS_PER_BLOCK // _PAD, _PAD), lambda i: (i, 0)),
        ],
        out_specs=pl.BlockSpec((_ROWS_PER_BLOCK, _OUT), lambda i: (i, 0)),
        out_shape=jax.ShapeDtypeStruct((R, _OUT), jnp.float32),
    )(*([x] * _NSPLIT), *([Wp] * _NSPLIT), bp, dv, idx)

    return out.reshape(B, T, _OUT)


# speedup vs baseline: 1.0795x; 1.0795x over previous
import functools

import jax
import jax.numpy as jnp
import numpy as np
from jax.experimental import pallas as pl

_ENUMS = (1, 2, 3, 4, 5)
_OUT = 73
_PAD = 128
_ROWS_PER_BLOCK = 4096
_NSPLIT = 2


def _mm_kernel(*refs):
    x_refs = refs[:_NSPLIT]
    w_refs = refs[_NSPLIT:2 * _NSPLIT]
    b_ref, dv_ref, idx_ref, o_ref = refs[2 * _NSPLIT:]
    dn = (((1,), (1,)), ((), ()))
    acc = jax.lax.dot_general(x_refs[0][...], w_refs[0][...], dn,
                              preferred_element_type=jnp.float32)
    for j in range(1, _NSPLIT):
        acc = acc + jax.lax.dot_general(x_refs[j][...], w_refs[j][...], dn,
                                        preferred_element_type=jnp.float32)
    acc = acc + b_ref[...]
    idxp = idx_ref[...]
    idx3 = jax.lax.broadcast_in_dim(
        idxp, (_ROWS_PER_BLOCK // _PAD, _PAD, _PAD), (0, 1))
    idx_col = idx3.reshape(_ROWS_PER_BLOCK, _PAD)
    mask = idx_col == dv_ref[...]
    o_ref[...] = jnp.where(mask, acc, 0.0)[:, :_OUT]


@jax.jit
def kernel(output_latents, output_decoder_index,
           W0, b0, W1, b1, W2, b2, W3, b3, W4, b4):
    B, T, D = output_latents.shape
    R = B * T
    Dc = D // _NSPLIT
    x = output_latents.reshape(R, D)
    idx = output_decoder_index.reshape(R // _PAD, _PAD)

    Wcat = jnp.concatenate([W0, W1, W2, W3, W4], axis=0)
    Wp = jnp.zeros((_PAD, D), jnp.float32).at[:_OUT].set(Wcat)
    bcat = jnp.concatenate([b0, b1, b2, b3, b4], axis=0)
    bp = jnp.zeros((1, _PAD), jnp.float32).at[0, :_OUT].set(bcat)

    dims = [2, 2, 2, 3, 64]
    dv_np = np.full((1, _PAD), -1, np.int32)
    off = 0
    for e, d in zip(_ENUMS, dims):
        dv_np[0, off:off + d] = e
        off += d
    dv = jnp.asarray(dv_np)

    grid = (R // _ROWS_PER_BLOCK,)
    x_specs = [
        pl.BlockSpec((_ROWS_PER_BLOCK, Dc),
                     functools.partial(lambda j, i: (i, j), j))
        for j in range(_NSPLIT)
    ]
    w_specs = [
        pl.BlockSpec((_PAD, Dc),
                     functools.partial(lambda j, i: (0, j), j))
        for j in range(_NSPLIT)
    ]
    out = pl.pallas_call(
        _mm_kernel,
        grid=grid,
        in_specs=x_specs + w_specs + [
            pl.BlockSpec((1, _PAD), lambda i: (0, 0)),
            pl.BlockSpec((1, _PAD), lambda i: (0, 0)),
            pl.BlockSpec((_ROWS_PER_BLOCK // _PAD, _PAD), lambda i: (i, 0)),
        ],
        out_specs=pl.BlockSpec((_ROWS_PER_BLOCK, _OUT), lambda i: (i, 0)),
        out_shape=jax.ShapeDtypeStruct((R, _OUT), jnp.float32),
    )(*([x] * _NSPLIT), *([Wp] * _NSPLIT), bp, dv, idx)

    return out.reshape(B, T, _OUT)
